# speedup vs baseline: 2.4811x; 2.4811x over previous
import functools

import jax
import jax.numpy as jnp
from jax import lax
from jax.experimental import pallas as pl
from jax.experimental.pallas import tpu as pltpu
from jax.experimental.pallas import tpu_sc as plsc


def _make_sc_gather(B, D, NC, NS):
    NW = NC * NS
    per_w = B // NW
    CH = 128
    n_ch = per_w // CH
    K8 = D // 16

    mesh = plsc.VectorSubcoreMesh(core_axis_name="c", subcore_axis_name="s")

    @functools.partial(
        pl.kernel,
        out_type=(
            jax.ShapeDtypeStruct((B, D), jnp.float32),
            jax.ShapeDtypeStruct((B, D), jnp.float32),
            jax.ShapeDtypeStruct((B, 16), jnp.float32),
        ),
        mesh=mesh,
        scratch_types=(
            pltpu.VMEM((CH,), jnp.int32),
            pltpu.VMEM((CH,), jnp.int32),
            pltpu.VMEM((CH, D), jnp.float32),
            pltpu.VMEM((CH, D), jnp.float32),
            pltpu.VMEM((CH, D), jnp.float32),
            pltpu.VMEM((CH, D), jnp.float32),
            pltpu.VMEM((D,), jnp.float32),
            pltpu.VMEM((CH, 16), jnp.float32),
            pltpu.SemaphoreType.DMA,
            pltpu.SemaphoreType.DMA,
            pltpu.SemaphoreType.DMA,
            pltpu.SemaphoreType.DMA,
            pltpu.SemaphoreType.DMA,
        ),
    )
    def sc_gather(users_hbm, items_hbm, gu_t, gi_t, mu_t, mi_t, wn_hbm,
                  mlp_u_out, mlp_i_out, gmfp_out,
                  uidx, iidx, gu, gi, mu, mi, wnv, gmfp,
                  sem_g0, sem_g1, sem_m0, sem_m1, sem_out):
        wid = lax.axis_index("s") * NC + lax.axis_index("c")
        pltpu.sync_copy(wn_hbm, wnv)

        for c in range(n_ch):
            base = wid * per_w + c * CH
            pltpu.sync_copy(users_hbm.at[pl.ds(base, CH)], uidx)
            pltpu.sync_copy(items_hbm.at[pl.ds(base, CH)], iidx)
            cp_g0 = pltpu.async_copy(gu_t.at[uidx], gu, sem_g0)
            cp_g1 = pltpu.async_copy(gi_t.at[iidx], gi, sem_g1)
            cp_m0 = pltpu.async_copy(mu_t.at[uidx], mu, sem_m0)
            cp_m1 = pltpu.async_copy(mi_t.at[iidx], mi, sem_m1)

            cp_m0.wait()
            out_m0 = pltpu.async_copy(mu, mlp_u_out.at[pl.ds(base, CH)], sem_out)
            cp_m1.wait()
            out_m1 = pltpu.async_copy(mi, mlp_i_out.at[pl.ds(base, CH)], sem_out)

            cp_g0.wait()
            cp_g1.wait()

            def row_body(i, carry):
                acc = gu[i, pl.ds(0, 16)] * gi[i, pl.ds(0, 16)] * wnv[pl.ds(0, 16)]
                for k in range(1, K8):
                    acc = acc + (gu[i, pl.ds(k * 16, 16)]
                                 * gi[i, pl.ds(k * 16, 16)]
                                 * wnv[pl.ds(k * 16, 16)])
                gmfp[i, :] = acc
                return carry

            lax.fori_loop(0, CH, row_body, 0)
            pltpu.sync_copy(gmfp, gmfp_out.at[pl.ds(base, CH)])
            out_m0.wait()
            out_m1.wait()

    return sc_gather


def _tc_mlp_body(mu_ref, mi_ref, gmfp_ref, w1u_ref, w1i_ref, b1_ref,
                 w2_ref, b2_ref, w3_ref, b3_ref, wnm_ref, bn_ref, out_ref):
    h = jnp.dot(mu_ref[...], w1u_ref[...], preferred_element_type=jnp.float32)
    h = h + jnp.dot(mi_ref[...], w1i_ref[...], preferred_element_type=jnp.float32)
    h = jax.nn.relu(h + b1_ref[...])
    h = jax.nn.relu(jnp.dot(h, w2_ref[...], preferred_element_type=jnp.float32)
                    + b2_ref[...])
    h = jax.nn.relu(jnp.dot(h, w3_ref[...], preferred_element_type=jnp.float32)
                    + b3_ref[...])
    logit = jnp.dot(h, wnm_ref[...], preferred_element_type=jnp.float32)
    logit = logit + jnp.sum(gmfp_ref[...], axis=1, keepdims=True) + bn_ref[...]
    out_ref[...] = 1.0 / (1.0 + jnp.exp(-logit))


def _make_tc_mlp(B, D, H1, H2, H3):
    BLK = 2048
    grid = (B // BLK,)
    full = lambda shape: pl.BlockSpec(shape, lambda i: (0, 0))
    return pl.pallas_call(
        _tc_mlp_body,
        grid=grid,
        in_specs=[
            pl.BlockSpec((BLK, D), lambda i: (i, 0)),
            pl.BlockSpec((BLK, D), lambda i: (i, 0)),
            pl.BlockSpec((BLK, 16), lambda i: (i, 0)),
            full((D, H1)),
            full((D, H1)),
            full((1, H1)),
            full((H1, H2)),
            full((1, H2)),
            full((H2, H3)),
            full((1, H3)),
            full((H3, 1)),
            full((1, 1)),
        ],
        out_specs=pl.BlockSpec((BLK, 1), lambda i: (i, 0)),
        out_shape=jax.ShapeDtypeStruct((B, 1), jnp.float32),
    )


def kernel(gmf_user_table, gmf_item_table, mlp_user_table, mlp_item_table,
           W1, b1, W2, b2, W3, b3, Wn, bn, users, items):
    B = users.shape[0]
    D = gmf_user_table.shape[1]
    H1, H2, H3 = W1.shape[1], W2.shape[1], W3.shape[1]

    info = plsc.get_sparse_core_info()
    NC, NS = info.num_cores, info.num_subcores

    users = users.astype(jnp.int32)
    items = items.astype(jnp.int32)
    wn_g = Wn[:D, 0]
    wn_m = Wn[D:, :]

    sc = _make_sc_gather(B, D, NC, NS)
    mlp_u_g, mlp_i_g, gmfp = sc(users, items,
                                gmf_user_table, gmf_item_table,
                                mlp_user_table, mlp_item_table, wn_g)

    tc = _make_tc_mlp(B, D, H1, H2, H3)
    pred = tc(mlp_u_g, mlp_i_g, gmfp,
              W1[:D], W1[D:], b1.reshape(1, H1),
              W2, b2.reshape(1, H2),
              W3, b3.reshape(1, H3),
              wn_m, bn.reshape(1, 1))
    return pred[:, 0]


# speedup vs baseline: 2.5122x; 1.0125x over previous
import functools

import jax
import jax.numpy as jnp
from jax import lax
from jax.experimental import pallas as pl
from jax.experimental.pallas import tpu as pltpu
from jax.experimental.pallas import tpu_sc as plsc


def _make_sc_gather(B, D, NC, NS):
    NW = NC * NS
    per_w = B // NW
    CH = 64
    n_ch = per_w // CH
    K8 = D // 16

    mesh = plsc.VectorSubcoreMesh(core_axis_name="c", subcore_axis_name="s")

    def buf_set():
        return (
            pltpu.VMEM((CH,), jnp.int32),
            pltpu.VMEM((CH,), jnp.int32),
            pltpu.VMEM((CH, D), jnp.float32),
            pltpu.VMEM((CH, D), jnp.float32),
            pltpu.VMEM((CH, D), jnp.float32),
            pltpu.VMEM((CH, D), jnp.float32),
            pltpu.VMEM((CH, 16), jnp.float32),
            pltpu.SemaphoreType.DMA,
            pltpu.SemaphoreType.DMA,
        )

    @functools.partial(
        pl.kernel,
        out_type=(
            jax.ShapeDtypeStruct((B, D), jnp.float32),
            jax.ShapeDtypeStruct((B, D), jnp.float32),
            jax.ShapeDtypeStruct((B, 16), jnp.float32),
        ),
        mesh=mesh,
        scratch_types=(
            pltpu.VMEM((D,), jnp.float32),
        ) + buf_set() + buf_set(),
    )
    def sc_gather(users_hbm, items_hbm, gu_t, gi_t, mu_t, mi_t, wn_hbm,
                  mlp_u_out, mlp_i_out, gmfp_out, wnv, *bufs):
        wid = lax.axis_index("s") * NC + lax.axis_index("c")
        pltpu.sync_copy(wn_hbm, wnv)
        sets = (bufs[:9], bufs[9:])

        def issue(c):
            uidx, iidx, gu, gi, mu, mi, _, sem_g, _ = sets[c % 2]
            base = wid * per_w + c * CH
            pltpu.sync_copy(users_hbm.at[pl.ds(base, CH)], uidx)
            pltpu.sync_copy(items_hbm.at[pl.ds(base, CH)], iidx)
            return (pltpu.async_copy(gu_t.at[uidx], gu, sem_g),
                    pltpu.async_copy(gi_t.at[iidx], gi, sem_g),
                    pltpu.async_copy(mu_t.at[uidx], mu, sem_g),
                    pltpu.async_copy(mi_t.at[iidx], mi, sem_g))

        pending = {0: issue(0)}
        if n_ch > 1:
            pending[1] = issue(1)
        outcps = {}

        for c in range(n_ch):
            _, _, gu, gi, mu, mi, gmfp, _, sem_o = sets[c % 2]
            base = wid * per_w + c * CH
            for cp in pending.pop(c):
                cp.wait()
            o1 = pltpu.async_copy(mu, mlp_u_out.at[pl.ds(base, CH)], sem_o)
            o2 = pltpu.async_copy(mi, mlp_i_out.at[pl.ds(base, CH)], sem_o)

            def row_body(i, carry):
                acc = gu[i, pl.ds(0, 16)] * gi[i, pl.ds(0, 16)] * wnv[pl.ds(0, 16)]
                for k in range(1, K8):
                    acc = acc + (gu[i, pl.ds(k * 16, 16)]
                                 * gi[i, pl.ds(k * 16, 16)]
                                 * wnv[pl.ds(k * 16, 16)])
                gmfp[i, :] = acc
                return carry

            lax.fori_loop(0, CH, row_body, 0)
            o3 = pltpu.async_copy(gmfp, gmfp_out.at[pl.ds(base, CH)], sem_o)
            outcps[c] = (o1, o2, o3)

            nxt = c + 2
            if nxt < n_ch:
                for cp in outcps.pop(c):
                    cp.wait()
                pending[nxt] = issue(nxt)

        for c in sorted(outcps):
            for cp in outcps[c]:
                cp.wait()

    return sc_gather


def _tc_mlp_body(mu_ref, mi_ref, gmfp_ref, w1u_ref, w1i_ref, b1_ref,
                 w2_ref, b2_ref, w3_ref, b3_ref, wnm_ref, bn_ref, out_ref):
    h = jnp.dot(mu_ref[...], w1u_ref[...], preferred_element_type=jnp.float32)
    h = h + jnp.dot(mi_ref[...], w1i_ref[...], preferred_element_type=jnp.float32)
    h = jax.nn.relu(h + b1_ref[...])
    h = jax.nn.relu(jnp.dot(h, w2_ref[...], preferred_element_type=jnp.float32)
                    + b2_ref[...])
    h = jax.nn.relu(jnp.dot(h, w3_ref[...], preferred_element_type=jnp.float32)
                    + b3_ref[...])
    logit = jnp.sum(h * wnm_ref[...], axis=1)
    logit = logit + jnp.sum(gmfp_ref[...], axis=1) + bn_ref[0, 0]
    out_ref[...] = 1.0 / (1.0 + jnp.exp(-logit))


def _make_tc_mlp(B, D, H1, H2, H3):
    BLK = 2048
    grid = (B // BLK,)
    full = lambda shape: pl.BlockSpec(shape, lambda i: (0, 0))
    return pl.pallas_call(
        _tc_mlp_body,
        grid=grid,
        in_specs=[
            pl.BlockSpec((BLK, D), lambda i: (i, 0)),
            pl.BlockSpec((BLK, D), lambda i: (i, 0)),
            pl.BlockSpec((BLK, 16), lambda i: (i, 0)),
            full((D, H1)),
            full((D, H1)),
            full((1, H1)),
            full((H1, H2)),
            full((1, H2)),
            full((H2, H3)),
            full((1, H3)),
            full((1, H3)),
            full((1, 1)),
        ],
        out_specs=pl.BlockSpec((BLK,), lambda i: (i,)),
        out_shape=jax.ShapeDtypeStruct((B,), jnp.float32),
    )


def kernel(gmf_user_table, gmf_item_table, mlp_user_table, mlp_item_table,
           W1, b1, W2, b2, W3, b3, Wn, bn, users, items):
    B = users.shape[0]
    D = gmf_user_table.shape[1]
    H1, H2, H3 = W1.shape[1], W2.shape[1], W3.shape[1]

    info = plsc.get_sparse_core_info()
    NC, NS = info.num_cores, info.num_subcores

    users = users.astype(jnp.int32)
    items = items.astype(jnp.int32)
    wn_g = Wn[:D, 0]
    wn_m = Wn[D:, :].reshape(1, H3)

    n_split = 2
    H = B // n_split
    sc = _make_sc_gather(H, D, NC, NS)
    tc = _make_tc_mlp(H, D, H1, H2, H3)

    sc_outs = [sc(users[i * H:(i + 1) * H], items[i * H:(i + 1) * H],
                  gmf_user_table, gmf_item_table,
                  mlp_user_table, mlp_item_table, wn_g)
               for i in range(n_split)]
    preds = [tc(mlp_u_g, mlp_i_g, gmfp,
                W1[:D], W1[D:], b1.reshape(1, H1),
                W2, b2.reshape(1, H2),
                W3, b3.reshape(1, H3),
                wn_m, bn.reshape(1, 1))
             for (mlp_u_g, mlp_i_g, gmfp) in sc_outs]
    return jnp.concatenate(preds, axis=0)
